# Initial kernel scaffold; baseline (speedup 1.0000x reference)
import jax
import jax.numpy as jnp
from jax.experimental import pallas as pl


def kernel(logits, targets, edge_batch, num_graphs):
    raise NotImplementedError("write your pallas kernel here")


# speedup vs baseline: 100.1362x; 100.1362x over previous
import functools

import jax
import jax.numpy as jnp
from jax import lax
from jax.experimental import pallas as pl
from jax.experimental.pallas import tpu as pltpu
from jax.experimental.pallas import tpu_sc as plsc

N = 3_200_000
G = 4096
TEMP = 0.07
INV_TEMP = float(1.0 / TEMP)
W_INF = 1.0
W_BCE = 0.5

NC = 2
NS = 16
NW = NC * NS
CHUNK = N // NW
SUB = 4000
NSUB = CHUNK // SUB
NV = SUB // 16

ACC_R, ACC_C = 16, 2048

_LOG1P_C = (
    0.9999999705406564, -0.499995015167874, 0.333192713425025,
    -0.24844369892463897, 0.19111431080531266, -0.13674769705378478,
    0.07836166801651114, -0.029588507391321184, 0.005253457796589797,
)

_mesh = plsc.VectorSubcoreMesh(core_axis_name="c", subcore_axis_name="s")


def _worker_id():
    return lax.axis_index("s") * NC + lax.axis_index("c")


@functools.partial(
    pl.kernel,
    mesh=_mesh,
    out_type=jax.ShapeDtypeStruct((NW, 16), jnp.float32),
    scratch_types=[
        pltpu.VMEM((SUB,), jnp.float32),
        pltpu.VMEM((16,), jnp.float32),
    ],
    compiler_params=pltpu.CompilerParams(needs_layout_passes=False),
)
def _max_pass(logits_hbm, out_hbm, buf, stage):
    wid = _worker_id()
    base = wid * CHUNK

    def outer(j, m):
        pltpu.sync_copy(logits_hbm.at[pl.ds(base + j * SUB, SUB)], buf)

        def inner(i, m):
            return jnp.maximum(m, buf[pl.ds(i * 16, 16)])

        return lax.fori_loop(0, NV, inner, m)

    m = lax.fori_loop(0, NSUB, outer, jnp.full((16,), -jnp.inf, jnp.float32))
    stage[...] = m
    pltpu.sync_copy(stage, out_hbm.at[wid])


@functools.partial(
    pl.kernel,
    mesh=_mesh,
    out_type=jax.ShapeDtypeStruct((NW, ACC_R * ACC_C), jnp.float32),
    scratch_types=[
        pltpu.VMEM((SUB,), jnp.float32),
        pltpu.VMEM((SUB,), jnp.float32),
        pltpu.VMEM((SUB,), jnp.int32),
        pltpu.VMEM((ACC_R * ACC_C,), jnp.float32),
        pltpu.VMEM((16,), jnp.float32),
    ],
    compiler_params=pltpu.CompilerParams(needs_layout_passes=False),
)
def _sum_pass(logits_hbm, targets_hbm, edges_hbm, gmax_hbm, out_hbm,
              lbuf, tbuf, ebuf, acc, gbuf):
    wid = _worker_id()
    base = wid * CHUNK

    z = jnp.zeros((16,), jnp.float32)

    def zero_body(c, _):
        acc[pl.ds(c * 16, 16)] = z
        return 0

    lax.fori_loop(0, (ACC_R * ACC_C) // 16, zero_body, 0)

    pltpu.sync_copy(gmax_hbm, gbuf)
    gmax = gbuf[...]

    ones = jnp.ones((16,), jnp.float32)

    def outer(j, _):
        off = base + j * SUB
        pltpu.sync_copy(logits_hbm.at[pl.ds(off, SUB)], lbuf)
        pltpu.sync_copy(targets_hbm.at[pl.ds(off, SUB)], tbuf)
        pltpu.sync_copy(edges_hbm.at[pl.ds(off, SUB)], ebuf)

        def inner(i, _):
            sl = pl.ds(i * 16, 16)
            l = lbuf[sl]
            t = tbuf[sl]
            seg = jnp.minimum(ebuf[sl], G - 1)
            s = l * jnp.float32(INV_TEMP)
            e_all = jnp.exp(s - gmax)
            pos = t > jnp.float32(0.5)
            e_pos = jnp.where(pos, e_all, jnp.float32(0.0))
            posf = jnp.where(pos, jnp.float32(1.0), jnp.float32(0.0))
            u = jnp.exp(-jnp.abs(l))
            p = jnp.float32(_LOG1P_C[-1])
            for c in _LOG1P_C[-2::-1]:
                p = p * u + jnp.float32(c)
            bce = jnp.maximum(l, jnp.float32(0.0)) - l * t + u * p
            plsc.addupdate_scatter(acc, [seg], e_all)
            plsc.addupdate_scatter(acc, [seg + G], e_pos)
            plsc.addupdate_scatter(acc, [seg + 2 * G], posf)
            plsc.addupdate_scatter(acc, [seg + 3 * G], ones)
            plsc.addupdate_scatter(acc, [seg + 4 * G], bce)
            return 0

        lax.fori_loop(0, NV, inner, 0)
        return 0

    lax.fori_loop(0, NSUB, outer, 0)
    pltpu.sync_copy(acc, out_hbm.at[wid])


def _finish_body(parts_ref, out_ref):
    p = jnp.sum(parts_ref[...], axis=0)
    sum_all = p[0:2]
    sum_pos = p[2:4]
    pos_counts = p[4:6]
    edge_counts = p[6:8]
    bce_sums = p[8:10]
    neg_counts = edge_counts - pos_counts
    valid = (pos_counts > 0) & (neg_counts > 0)
    diff = jnp.where(valid, jnp.log(sum_all) - jnp.log(sum_pos), 0.0)
    n_valid = jnp.maximum(jnp.sum(valid.astype(jnp.float32)), 1.0)
    loss_inf = jnp.sum(diff) / n_valid
    valid_b = edge_counts > 0
    per_graph = jnp.where(valid_b, bce_sums / jnp.maximum(edge_counts, 1.0), 0.0)
    n_valid_b = jnp.maximum(jnp.sum(valid_b.astype(jnp.float32)), 1.0)
    loss_bce = jnp.sum(per_graph) / n_valid_b
    out_ref[0, 0] = W_INF * loss_inf + W_BCE * loss_bce


def kernel(logits, targets, edge_batch, num_graphs):
    del num_graphs
    logits = logits.reshape(-1).astype(jnp.float32)
    targets = targets.reshape(-1).astype(jnp.float32)
    edge_batch = edge_batch.reshape(-1).astype(jnp.int32)
    maxes = _max_pass(logits)
    gmax = jnp.full((16,), jnp.max(maxes) * jnp.float32(INV_TEMP), jnp.float32)
    parts = _sum_pass(logits, targets, edge_batch, gmax)
    parts = parts.reshape(NW, ACC_R, ACC_C)
    out = pl.pallas_call(
        _finish_body,
        out_shape=jax.ShapeDtypeStruct((1, 1), jnp.float32),
        out_specs=pl.BlockSpec(memory_space=pltpu.SMEM),
    )(parts)
    return out[0, 0]


# speedup vs baseline: 208.8284x; 2.0854x over previous
import functools

import jax
import jax.numpy as jnp
from jax import lax
from jax.experimental import pallas as pl
from jax.experimental.pallas import tpu as pltpu
from jax.experimental.pallas import tpu_sc as plsc

N = 3_200_000
G = 4096
TEMP = 0.07
INV_TEMP = float(1.0 / TEMP)
W_INF = 1.0
W_BCE = 0.5

NC = 2
NS = 16
NW = NC * NS
CHUNK = N // NW
SUB = 4000
NSUB = CHUNK // SUB
NV = SUB // 16

ACC_R, ACC_C = 10, 2048

_LOG1P_C = (
    0.9999999705406564, -0.499995015167874, 0.333192713425025,
    -0.24844369892463897, 0.19111431080531266, -0.13674769705378478,
    0.07836166801651114, -0.029588507391321184, 0.005253457796589797,
)

_mesh = plsc.VectorSubcoreMesh(core_axis_name="c", subcore_axis_name="s")


def _worker_id():
    return lax.axis_index("s") * NC + lax.axis_index("c")


@functools.partial(
    pl.kernel,
    mesh=_mesh,
    out_type=jax.ShapeDtypeStruct((NW, 16), jnp.float32),
    scratch_types=[
        pltpu.VMEM((SUB,), jnp.float32),
        pltpu.VMEM((16,), jnp.float32),
    ],
    compiler_params=pltpu.CompilerParams(needs_layout_passes=False),
)
def _max_pass(logits_hbm, out_hbm, buf, stage):
    wid = _worker_id()
    base = wid * CHUNK

    def outer(j, m):
        pltpu.sync_copy(logits_hbm.at[pl.ds(base + j * SUB, SUB)], buf)

        def inner(i, m):
            return jnp.maximum(m, buf[pl.ds(i * 16, 16)])

        return lax.fori_loop(0, NV, inner, m)

    m = lax.fori_loop(0, NSUB, outer, jnp.full((16,), -jnp.inf, jnp.float32))
    stage[...] = m
    pltpu.sync_copy(stage, out_hbm.at[wid])


@functools.partial(
    pl.kernel,
    mesh=_mesh,
    out_type=jax.ShapeDtypeStruct((NW, ACC_R * ACC_C), jnp.float32),
    scratch_types=[
        pltpu.VMEM((SUB,), jnp.float32),
        pltpu.VMEM((SUB,), jnp.float32),
        pltpu.VMEM((SUB,), jnp.int32),
        pltpu.VMEM((ACC_R * ACC_C,), jnp.float32),
        pltpu.VMEM((16,), jnp.float32),
    ],
    compiler_params=pltpu.CompilerParams(needs_layout_passes=False),
)
def _sum_pass(logits_hbm, targets_hbm, edges_hbm, gmax_hbm, out_hbm,
              lbuf, tbuf, ebuf, acc, gbuf):
    wid = _worker_id()
    base = wid * CHUNK

    z = jnp.zeros((16,), jnp.float32)

    def zero_body(c, _):
        acc[pl.ds(c * 16, 16)] = z
        return 0

    lax.fori_loop(0, (ACC_R * ACC_C) // 16, zero_body, 0)

    pltpu.sync_copy(gmax_hbm, gbuf)
    gmax = gbuf[...]

    ones = jnp.ones((16,), jnp.float32)
    zf = jnp.zeros((16,), jnp.float32)

    def flush(cur_f, a0, a1, a2, a3, a4):
        cur = cur_f.astype(jnp.int32)
        plsc.addupdate_scatter(acc, [cur], a0)
        plsc.addupdate_scatter(acc, [cur + G], a1)
        plsc.addupdate_scatter(acc, [cur + 2 * G], a2)
        plsc.addupdate_scatter(acc, [cur + 3 * G], a3)
        plsc.addupdate_scatter(acc, [cur + 4 * G], a4)

    c15 = jnp.full((16, 1), 15, jnp.int32)
    _dnums = lax.GatherDimensionNumbers(
        offset_dims=(), collapsed_slice_dims=(0,), start_index_map=(0,))

    def _splat_last(x):
        return lax.gather(x, c15, _dnums, (1,),
                          mode=lax.GatherScatterMode.PROMISE_IN_BOUNDS)

    def outer(j, carry):
        off = base + j * SUB
        pltpu.sync_copy(logits_hbm.at[pl.ds(off, SUB)], lbuf)
        pltpu.sync_copy(targets_hbm.at[pl.ds(off, SUB)], tbuf)
        pltpu.sync_copy(edges_hbm.at[pl.ds(off, SUB)], ebuf)

        def inner(i, carry):
            cur_f, a0, a1, a2, a3, a4 = carry
            sl = pl.ds(i * 16, 16)
            l = lbuf[sl]
            t = tbuf[sl]
            seg = jnp.minimum(ebuf[sl], G - 1)
            seg_f = seg.astype(jnp.float32)
            s = l * jnp.float32(INV_TEMP)
            e_all = jnp.exp(s - gmax)
            pos = t > jnp.float32(0.5)
            e_pos = jnp.where(pos, e_all, jnp.float32(0.0))
            posf = jnp.where(pos, jnp.float32(1.0), jnp.float32(0.0))
            u = jnp.exp(-jnp.abs(l))
            p = jnp.float32(_LOG1P_C[-1])
            for c in _LOG1P_C[-2::-1]:
                p = p * u + jnp.float32(c)
            bce = jnp.maximum(l, jnp.float32(0.0)) - l * t + u * p

            cur_new = _splat_last(seg_f)
            m_prev = seg_f == cur_f
            chg = cur_new != cur_f
            m_new = seg_f == cur_new
            mid = (~m_prev) & (~m_new)
            cur_i = cur_f.astype(jnp.int32)
            zero = jnp.float32(0.0)

            def step(aq, v, k):
                aq1 = aq + jnp.where(m_prev, v, zero)
                plsc.addupdate_scatter(acc, [cur_i + k * G], aq1, mask=chg)
                plsc.addupdate_scatter(acc, [seg + k * G], v, mask=mid)
                return jnp.where(chg, jnp.where(m_new, v, zero), aq1)

            a0 = step(a0, e_all, 0)
            a1 = step(a1, e_pos, 1)
            a2 = step(a2, posf, 2)
            a3 = step(a3, ones, 3)
            a4 = step(a4, bce, 4)
            return cur_new, a0, a1, a2, a3, a4

        return lax.fori_loop(0, NV, inner, carry)

    cur0 = jnp.zeros((16,), jnp.float32)
    carry = lax.fori_loop(0, NSUB, outer, (cur0, zf, zf, zf, zf, zf))
    flush(*carry)
    pltpu.sync_copy(acc, out_hbm.at[wid])


def _finish_body(parts_ref, out_ref):
    p = jnp.sum(parts_ref[...], axis=0)
    sum_all = p[0:2]
    sum_pos = p[2:4]
    pos_counts = p[4:6]
    edge_counts = p[6:8]
    bce_sums = p[8:10]
    valid = (pos_counts > 0) & (edge_counts - pos_counts > 0)
    diff = jnp.where(valid, jnp.log(sum_all) - jnp.log(sum_pos), 0.0)
    n_valid = jnp.maximum(jnp.sum(valid.astype(jnp.float32)), 1.0)
    loss_inf = jnp.sum(diff) / n_valid
    valid_b = edge_counts > 0
    per_graph = jnp.where(valid_b, bce_sums / jnp.maximum(edge_counts, 1.0), 0.0)
    n_valid_b = jnp.maximum(jnp.sum(valid_b.astype(jnp.float32)), 1.0)
    loss_bce = jnp.sum(per_graph) / n_valid_b
    out_ref[0, 0] = W_INF * loss_inf + W_BCE * loss_bce


def kernel(logits, targets, edge_batch, num_graphs):
    del num_graphs
    logits = logits.reshape(-1).astype(jnp.float32)
    targets = targets.reshape(-1).astype(jnp.float32)
    edge_batch = edge_batch.reshape(-1).astype(jnp.int32)
    maxes = _max_pass(logits)
    gmax = jnp.full((16,), jnp.max(maxes) * jnp.float32(INV_TEMP), jnp.float32)
    parts = _sum_pass(logits, targets, edge_batch, gmax)
    parts = parts.reshape(NW, ACC_R, ACC_C)
    out = pl.pallas_call(
        _finish_body,
        out_shape=jax.ShapeDtypeStruct((1, 1), jnp.float32),
        out_specs=pl.BlockSpec(memory_space=pltpu.SMEM),
    )(parts)
    return out[0, 0]


# speedup vs baseline: 241.1981x; 1.1550x over previous
import functools

import jax
import jax.numpy as jnp
from jax import lax
from jax.experimental import pallas as pl
from jax.experimental.pallas import tpu as pltpu
from jax.experimental.pallas import tpu_sc as plsc

N = 3_200_000
G = 4096
TEMP = 0.07
INV_TEMP = float(1.0 / TEMP)
W_INF = 1.0
W_BCE = 0.5

NC = 2
NS = 16
NW = NC * NS
CHUNK = N // NW
SUB = 20000
NSUB = CHUNK // SUB
NV = SUB // 16

ACC_R, ACC_C = 10, 2048

_LOG1P_C = (
    0.9999999705406564, -0.499995015167874, 0.333192713425025,
    -0.24844369892463897, 0.19111431080531266, -0.13674769705378478,
    0.07836166801651114, -0.029588507391321184, 0.005253457796589797,
)

_mesh = plsc.VectorSubcoreMesh(core_axis_name="c", subcore_axis_name="s")


def _worker_id():
    return lax.axis_index("s") * NC + lax.axis_index("c")


@functools.partial(
    pl.kernel,
    mesh=_mesh,
    out_type=jax.ShapeDtypeStruct((NW, 16), jnp.float32),
    scratch_types=[
        pltpu.VMEM((SUB,), jnp.float32),
        pltpu.VMEM((16,), jnp.float32),
    ],
    compiler_params=pltpu.CompilerParams(needs_layout_passes=False),
)
def _max_pass(logits_hbm, out_hbm, buf, stage):
    wid = _worker_id()
    base = wid * CHUNK

    def outer(j, m):
        pltpu.sync_copy(logits_hbm.at[pl.ds(base + j * SUB, SUB)], buf)

        def inner(i, m):
            return jnp.maximum(m, buf[pl.ds(i * 16, 16)])

        return lax.fori_loop(0, NV, inner, m)

    m = lax.fori_loop(0, NSUB, outer, jnp.full((16,), -jnp.inf, jnp.float32))
    stage[...] = m
    pltpu.sync_copy(stage, out_hbm.at[wid])


@functools.partial(
    pl.kernel,
    mesh=_mesh,
    out_type=jax.ShapeDtypeStruct((NW, ACC_R * ACC_C), jnp.float32),
    scratch_types=[
        pltpu.VMEM((SUB,), jnp.float32),
        pltpu.VMEM((SUB,), jnp.float32),
        pltpu.VMEM((SUB,), jnp.int32),
        pltpu.VMEM((ACC_R * ACC_C,), jnp.float32),
        pltpu.VMEM((16,), jnp.float32),
    ],
    compiler_params=pltpu.CompilerParams(needs_layout_passes=False),
)
def _sum_pass(logits_hbm, targets_hbm, edges_hbm, gmax_hbm, out_hbm,
              lbuf, tbuf, ebuf, acc, gbuf):
    wid = _worker_id()
    base = wid * CHUNK

    z = jnp.zeros((16,), jnp.float32)

    def zero_body(c, _):
        acc[pl.ds(c * 16, 16)] = z
        return 0

    lax.fori_loop(0, (ACC_R * ACC_C) // 16, zero_body, 0)

    pltpu.sync_copy(gmax_hbm, gbuf)
    gmax = gbuf[...]

    ones = jnp.ones((16,), jnp.float32)
    zf = jnp.zeros((16,), jnp.float32)

    def flush(cur_f, a0, a1, a2, a3, a4):
        cur = cur_f.astype(jnp.int32)
        plsc.addupdate_scatter(acc, [cur], a0)
        plsc.addupdate_scatter(acc, [cur + G], a1)
        plsc.addupdate_scatter(acc, [cur + 2 * G], a2)
        plsc.addupdate_scatter(acc, [cur + 3 * G], a3)
        plsc.addupdate_scatter(acc, [cur + 4 * G], a4)

    c15 = jnp.full((16, 1), 15, jnp.int32)
    _dnums = lax.GatherDimensionNumbers(
        offset_dims=(), collapsed_slice_dims=(0,), start_index_map=(0,))

    def _splat_last(x):
        return lax.gather(x, c15, _dnums, (1,),
                          mode=lax.GatherScatterMode.PROMISE_IN_BOUNDS)

    def outer(j, carry):
        off = base + j * SUB
        pltpu.sync_copy(logits_hbm.at[pl.ds(off, SUB)], lbuf)
        pltpu.sync_copy(targets_hbm.at[pl.ds(off, SUB)], tbuf)
        pltpu.sync_copy(edges_hbm.at[pl.ds(off, SUB)], ebuf)

        def body(i, carry):
            cur_f, a0, a1, a2, a3, a4 = carry
            sl = pl.ds(i * 16, 16)
            l = lbuf[sl]
            t = tbuf[sl]
            seg = jnp.minimum(ebuf[sl], G - 1)
            seg_f = seg.astype(jnp.float32)
            s = l * jnp.float32(INV_TEMP)
            e_all = jnp.exp(s - gmax)
            pos = t > jnp.float32(0.5)
            e_pos = jnp.where(pos, e_all, jnp.float32(0.0))
            posf = jnp.where(pos, jnp.float32(1.0), jnp.float32(0.0))
            u = jnp.exp(-jnp.abs(l))
            p = jnp.float32(_LOG1P_C[-1])
            for c in _LOG1P_C[-2::-1]:
                p = p * u + jnp.float32(c)
            bce = jnp.maximum(l, jnp.float32(0.0)) - l * t + u * p

            cur_new = _splat_last(seg_f)
            m_prev = seg_f == cur_f
            chg = cur_new != cur_f
            m_new = seg_f == cur_new
            mid = (~m_prev) & (~m_new)
            cur_i = cur_f.astype(jnp.int32)
            zero = jnp.float32(0.0)

            def step(aq, v, k):
                aq1 = aq + jnp.where(m_prev, v, zero)
                plsc.addupdate_scatter(acc, [cur_i + k * G], aq1, mask=chg)
                plsc.addupdate_scatter(acc, [seg + k * G], v, mask=mid)
                return jnp.where(chg, jnp.where(m_new, v, zero), aq1)

            a0 = step(a0, e_all, 0)
            a1 = step(a1, e_pos, 1)
            a2 = step(a2, posf, 2)
            a3 = step(a3, ones, 3)
            a4 = step(a4, bce, 4)
            return cur_new, a0, a1, a2, a3, a4

        def inner(i, carry):
            carry = body(2 * i, carry)
            return body(2 * i + 1, carry)

        return lax.fori_loop(0, NV // 2, inner, carry)

    cur0 = jnp.zeros((16,), jnp.float32)
    carry = lax.fori_loop(0, NSUB, outer, (cur0, zf, zf, zf, zf, zf))
    flush(*carry)
    pltpu.sync_copy(acc, out_hbm.at[wid])


def _finish_body(parts_ref, out_ref):
    p = jnp.sum(parts_ref[...], axis=0)
    sum_all = p[0:2]
    sum_pos = p[2:4]
    pos_counts = p[4:6]
    edge_counts = p[6:8]
    bce_sums = p[8:10]
    valid = (pos_counts > 0) & (edge_counts - pos_counts > 0)
    diff = jnp.where(valid, jnp.log(sum_all) - jnp.log(sum_pos), 0.0)
    n_valid = jnp.maximum(jnp.sum(valid.astype(jnp.float32)), 1.0)
    loss_inf = jnp.sum(diff) / n_valid
    valid_b = edge_counts > 0
    per_graph = jnp.where(valid_b, bce_sums / jnp.maximum(edge_counts, 1.0), 0.0)
    n_valid_b = jnp.maximum(jnp.sum(valid_b.astype(jnp.float32)), 1.0)
    loss_bce = jnp.sum(per_graph) / n_valid_b
    out_ref[0, 0] = W_INF * loss_inf + W_BCE * loss_bce


def kernel(logits, targets, edge_batch, num_graphs):
    del num_graphs
    logits = logits.reshape(-1).astype(jnp.float32)
    targets = targets.reshape(-1).astype(jnp.float32)
    edge_batch = edge_batch.reshape(-1).astype(jnp.int32)
    maxes = _max_pass(logits)
    gmax = jnp.full((16,), jnp.max(maxes) * jnp.float32(INV_TEMP), jnp.float32)
    parts = _sum_pass(logits, targets, edge_batch, gmax)
    parts = parts.reshape(NW, ACC_R, ACC_C)
    out = pl.pallas_call(
        _finish_body,
        out_shape=jax.ShapeDtypeStruct((1, 1), jnp.float32),
        out_specs=pl.BlockSpec(memory_space=pltpu.SMEM),
    )(parts)
    return out[0, 0]


# speedup vs baseline: 242.0357x; 1.0035x over previous
import functools

import jax
import jax.numpy as jnp
from jax import lax
from jax.experimental import pallas as pl
from jax.experimental.pallas import tpu as pltpu
from jax.experimental.pallas import tpu_sc as plsc

N = 3_200_000
G = 4096
TEMP = 0.07
INV_TEMP = float(1.0 / TEMP)
W_INF = 1.0
W_BCE = 0.5

NC = 2
NS = 16
NW = NC * NS
CHUNK = N // NW
SUB = 20000
NSUB = CHUNK // SUB
NV = SUB // 16

ACC_R, ACC_C = 10, 2048

_LOG1P_C = (
    0.9999999705406564, -0.499995015167874, 0.333192713425025,
    -0.24844369892463897, 0.19111431080531266, -0.13674769705378478,
    0.07836166801651114, -0.029588507391321184, 0.005253457796589797,
)

_mesh = plsc.VectorSubcoreMesh(core_axis_name="c", subcore_axis_name="s")


def _worker_id():
    return lax.axis_index("s") * NC + lax.axis_index("c")


@functools.partial(
    pl.kernel,
    mesh=_mesh,
    out_type=jax.ShapeDtypeStruct((NW, 16), jnp.float32),
    scratch_types=[
        pltpu.VMEM((SUB,), jnp.float32),
        pltpu.VMEM((16,), jnp.float32),
    ],
    compiler_params=pltpu.CompilerParams(needs_layout_passes=False),
)
def _max_pass(logits_hbm, out_hbm, buf, stage):
    wid = _worker_id()
    base = wid * CHUNK

    def outer(j, m):
        pltpu.sync_copy(logits_hbm.at[pl.ds(base + j * SUB, SUB)], buf)

        def inner(i, m):
            return jnp.maximum(m, buf[pl.ds(i * 16, 16)])

        return lax.fori_loop(0, NV, inner, m)

    m = lax.fori_loop(0, NSUB, outer, jnp.full((16,), -jnp.inf, jnp.float32))
    stage[...] = m
    pltpu.sync_copy(stage, out_hbm.at[wid])


@functools.partial(
    pl.kernel,
    mesh=_mesh,
    out_type=jax.ShapeDtypeStruct((NW, ACC_R * ACC_C), jnp.float32),
    scratch_types=[
        pltpu.VMEM((SUB,), jnp.float32),
        pltpu.VMEM((SUB,), jnp.float32),
        pltpu.VMEM((SUB,), jnp.int32),
        pltpu.VMEM((ACC_R * ACC_C,), jnp.float32),
        pltpu.VMEM((16,), jnp.float32),
    ],
    compiler_params=pltpu.CompilerParams(needs_layout_passes=False),
)
def _sum_pass(logits_hbm, targets_hbm, edges_hbm, gmax_hbm, out_hbm,
              lbuf, tbuf, ebuf, acc, gbuf):
    wid = _worker_id()
    base = wid * CHUNK

    z = jnp.zeros((16,), jnp.float32)

    def zero_body(c, _):
        acc[pl.ds(c * 16, 16)] = z
        return 0

    lax.fori_loop(0, (ACC_R * ACC_C) // 16, zero_body, 0)

    pltpu.sync_copy(gmax_hbm, gbuf)
    gmax = gbuf[...]

    ones = jnp.ones((16,), jnp.float32)
    zf = jnp.zeros((16,), jnp.float32)

    def flush(cur_f, a0, a1, a2, a3, a4):
        cur = cur_f.astype(jnp.int32)
        plsc.addupdate_scatter(acc, [cur], a0)
        plsc.addupdate_scatter(acc, [cur + G], a1)
        plsc.addupdate_scatter(acc, [cur + 2 * G], a2)
        plsc.addupdate_scatter(acc, [cur + 3 * G], a3)
        plsc.addupdate_scatter(acc, [cur + 4 * G], a4)

    c15 = jnp.full((16, 1), 15, jnp.int32)
    _dnums = lax.GatherDimensionNumbers(
        offset_dims=(), collapsed_slice_dims=(0,), start_index_map=(0,))

    def _splat_last(x):
        return lax.gather(x, c15, _dnums, (1,),
                          mode=lax.GatherScatterMode.PROMISE_IN_BOUNDS)

    def outer(j, carry):
        off = base + j * SUB
        pltpu.sync_copy(logits_hbm.at[pl.ds(off, SUB)], lbuf)
        pltpu.sync_copy(targets_hbm.at[pl.ds(off, SUB)], tbuf)
        pltpu.sync_copy(edges_hbm.at[pl.ds(off, SUB)], ebuf)

        def body(i, carry):
            cur_f, a0, a1, a2, a3, a4 = carry
            sl = pl.ds(i * 16, 16)
            l = lbuf[sl]
            t = tbuf[sl]
            seg = jnp.minimum(ebuf[sl], G - 1)
            seg_f = seg.astype(jnp.float32)
            s = l * jnp.float32(INV_TEMP)
            e_all = jnp.exp(s - gmax)
            pos = t > jnp.float32(0.5)
            e_pos = jnp.where(pos, e_all, jnp.float32(0.0))
            posf = jnp.where(pos, jnp.float32(1.0), jnp.float32(0.0))
            u = jnp.exp(-jnp.abs(l))
            p = jnp.float32(_LOG1P_C[-1])
            for c in _LOG1P_C[-2::-1]:
                p = p * u + jnp.float32(c)
            bce = jnp.maximum(l, jnp.float32(0.0)) - l * t + u * p

            cur_new = _splat_last(seg_f)
            m_prev = seg_f == cur_f
            chg = cur_new != cur_f
            m_new = seg_f == cur_new
            mid = (~m_prev) & (~m_new)
            cur_i = cur_f.astype(jnp.int32)
            zero = jnp.float32(0.0)

            def step(aq, v, k):
                aq1 = aq + jnp.where(m_prev, v, zero)
                plsc.addupdate_scatter(acc, [cur_i + k * G], aq1, mask=chg)
                plsc.addupdate_scatter(acc, [seg + k * G], v, mask=mid)
                return jnp.where(chg, jnp.where(m_new, v, zero), aq1)

            a0 = step(a0, e_all, 0)
            a1 = step(a1, e_pos, 1)
            a2 = step(a2, posf, 2)
            a3 = step(a3, ones, 3)
            a4 = step(a4, bce, 4)
            return cur_new, a0, a1, a2, a3, a4

        def inner(i, carry):
            for k in range(5):
                carry = body(5 * i + k, carry)
            return carry

        return lax.fori_loop(0, NV // 5, inner, carry)

    cur0 = jnp.zeros((16,), jnp.float32)
    carry = lax.fori_loop(0, NSUB, outer, (cur0, zf, zf, zf, zf, zf))
    flush(*carry)
    pltpu.sync_copy(acc, out_hbm.at[wid])


def _finish_body(parts_ref, out_ref):
    p = jnp.sum(parts_ref[...], axis=0)
    sum_all = p[0:2]
    sum_pos = p[2:4]
    pos_counts = p[4:6]
    edge_counts = p[6:8]
    bce_sums = p[8:10]
    valid = (pos_counts > 0) & (edge_counts - pos_counts > 0)
    diff = jnp.where(valid, jnp.log(sum_all) - jnp.log(sum_pos), 0.0)
    n_valid = jnp.maximum(jnp.sum(valid.astype(jnp.float32)), 1.0)
    loss_inf = jnp.sum(diff) / n_valid
    valid_b = edge_counts > 0
    per_graph = jnp.where(valid_b, bce_sums / jnp.maximum(edge_counts, 1.0), 0.0)
    n_valid_b = jnp.maximum(jnp.sum(valid_b.astype(jnp.float32)), 1.0)
    loss_bce = jnp.sum(per_graph) / n_valid_b
    out_ref[0, 0] = W_INF * loss_inf + W_BCE * loss_bce


def kernel(logits, targets, edge_batch, num_graphs):
    del num_graphs
    logits = logits.reshape(-1).astype(jnp.float32)
    targets = targets.reshape(-1).astype(jnp.float32)
    edge_batch = edge_batch.reshape(-1).astype(jnp.int32)
    maxes = _max_pass(logits)
    gmax = jnp.full((16,), jnp.max(maxes) * jnp.float32(INV_TEMP), jnp.float32)
    parts = _sum_pass(logits, targets, edge_batch, gmax)
    parts = parts.reshape(NW, ACC_R, ACC_C)
    out = pl.pallas_call(
        _finish_body,
        out_shape=jax.ShapeDtypeStruct((1, 1), jnp.float32),
        out_specs=pl.BlockSpec(memory_space=pltpu.SMEM),
    )(parts)
    return out[0, 0]


# speedup vs baseline: 400.0787x; 1.6530x over previous
import functools

import jax
import jax.numpy as jnp
from jax import lax
from jax.experimental import pallas as pl
from jax.experimental.pallas import tpu as pltpu
from jax.experimental.pallas import tpu_sc as plsc

N = 3_200_000
G = 4096
TEMP = 0.07
INV_TEMP = float(1.0 / TEMP)
W_INF = 1.0
W_BCE = 0.5

NC = 2
NS = 16
NW = NC * NS
CHUNK = N // NW
SUB = 20000
NSUB = CHUNK // SUB
NV = SUB // 16

ACC_R, ACC_C = 10, 2048

_LOG1P_C = (
    0.9999999705406564, -0.499995015167874, 0.333192713425025,
    -0.24844369892463897, 0.19111431080531266, -0.13674769705378478,
    0.07836166801651114, -0.029588507391321184, 0.005253457796589797,
)

_mesh = plsc.VectorSubcoreMesh(core_axis_name="c", subcore_axis_name="s")


def _worker_id():
    return lax.axis_index("s") * NC + lax.axis_index("c")


@functools.partial(
    pl.kernel,
    mesh=_mesh,
    out_type=jax.ShapeDtypeStruct((NW, 16), jnp.float32),
    scratch_types=[
        pltpu.VMEM((SUB,), jnp.float32),
        pltpu.VMEM((16,), jnp.float32),
    ],
    compiler_params=pltpu.CompilerParams(needs_layout_passes=False),
)
def _max_pass(logits_hbm, out_hbm, buf, stage):
    wid = _worker_id()
    base = wid * CHUNK

    def outer(j, m):
        pltpu.sync_copy(logits_hbm.at[pl.ds(base + j * SUB, SUB)], buf)

        def inner(i, m):
            return jnp.maximum(m, buf[pl.ds(i * 16, 16)])

        return lax.fori_loop(0, NV, inner, m)

    m = lax.fori_loop(0, NSUB, outer, jnp.full((16,), -jnp.inf, jnp.float32))
    stage[...] = m
    pltpu.sync_copy(stage, out_hbm.at[wid])


@functools.partial(
    pl.kernel,
    mesh=_mesh,
    out_type=jax.ShapeDtypeStruct((NW, ACC_R * ACC_C), jnp.float32),
    scratch_types=[
        pltpu.VMEM((SUB,), jnp.float32),
        pltpu.VMEM((SUB,), jnp.float32),
        pltpu.VMEM((SUB,), jnp.int32),
        pltpu.VMEM((ACC_R * ACC_C,), jnp.float32),
        pltpu.VMEM((16,), jnp.float32),
    ],
    compiler_params=pltpu.CompilerParams(needs_layout_passes=False),
)
def _sum_pass(logits_hbm, targets_hbm, edges_hbm, gmax_hbm, out_hbm,
              lbuf, tbuf, ebuf, acc, gbuf):
    wid = _worker_id()
    base = wid * CHUNK

    z = jnp.zeros((16,), jnp.float32)

    def zero_body(c, _):
        acc[pl.ds(c * 16, 16)] = z
        return 0

    lax.fori_loop(0, (ACC_R * ACC_C) // 16, zero_body, 0)

    pltpu.sync_copy(gmax_hbm, gbuf)
    gmax = gbuf[...]

    ones = jnp.ones((16,), jnp.float32)
    zf = jnp.zeros((16,), jnp.float32)

    def flush(cur_f, a0, a1, a2, a3, a4):
        cur = cur_f.astype(jnp.int32)
        plsc.addupdate_scatter(acc, [cur], a0)
        plsc.addupdate_scatter(acc, [cur + G], a1)
        plsc.addupdate_scatter(acc, [cur + 2 * G], a2)
        plsc.addupdate_scatter(acc, [cur + 3 * G], a3)
        plsc.addupdate_scatter(acc, [cur + 4 * G], a4)

    c15 = jnp.full((16, 1), 15, jnp.int32)
    _dnums = lax.GatherDimensionNumbers(
        offset_dims=(), collapsed_slice_dims=(0,), start_index_map=(0,))

    def _splat_last(x):
        return lax.gather(x, c15, _dnums, (1,),
                          mode=lax.GatherScatterMode.PROMISE_IN_BOUNDS)

    def outer(j, carry):
        off = base + j * SUB
        pltpu.sync_copy(logits_hbm.at[pl.ds(off, SUB)], lbuf)
        pltpu.sync_copy(targets_hbm.at[pl.ds(off, SUB)], tbuf)
        pltpu.sync_copy(edges_hbm.at[pl.ds(off, SUB)], ebuf)

        zero = jnp.float32(0.0)

        def compute(i):
            sl = pl.ds(i * 16, 16)
            l = lbuf[sl]
            t = tbuf[sl]
            seg = jnp.minimum(ebuf[sl], G - 1)
            seg_f = seg.astype(jnp.float32)
            s = l * jnp.float32(INV_TEMP)
            e_all = jnp.exp(s - gmax)
            pos = t > jnp.float32(0.5)
            e_pos = jnp.where(pos, e_all, zero)
            posf = jnp.where(pos, jnp.float32(1.0), zero)
            u = jnp.exp(-jnp.abs(l))
            p = jnp.float32(_LOG1P_C[-1])
            for c in _LOG1P_C[-2::-1]:
                p = p * u + jnp.float32(c)
            bce = jnp.maximum(l, jnp.float32(0.0)) - l * t + u * p
            return seg, seg_f, (e_all, e_pos, posf, ones, bce)

        UNROLL = 5

        def inner(i, carry):
            cur_f, *accs = carry
            datas = [compute(UNROLL * i + k) for k in range(UNROLL)]
            cur_new = _splat_last(datas[-1][1])
            chg = cur_new != cur_f
            cur_i = cur_f.astype(jnp.int32)
            aq1s = list(accs)
            news = [zf] * 5
            for seg, seg_f, vals in datas:
                m_prev = seg_f == cur_f
                m_new = seg_f == cur_new
                mid = (~m_prev) & (~m_new)
                for q in range(5):
                    aq1s[q] = aq1s[q] + jnp.where(m_prev, vals[q], zero)
                    news[q] = news[q] + jnp.where(m_new, vals[q], zero)
                    plsc.addupdate_scatter(acc, [seg + q * G], vals[q], mask=mid)
            outs = []
            for q in range(5):
                plsc.addupdate_scatter(acc, [cur_i + q * G], aq1s[q], mask=chg)
                outs.append(jnp.where(chg, news[q], aq1s[q]))
            return (cur_new, *outs)

        return lax.fori_loop(0, NV // UNROLL, inner, carry)

    cur0 = jnp.zeros((16,), jnp.float32)
    carry = lax.fori_loop(0, NSUB, outer, (cur0, zf, zf, zf, zf, zf))
    flush(*carry)
    pltpu.sync_copy(acc, out_hbm.at[wid])


def _finish_body(parts_ref, out_ref):
    p = jnp.sum(parts_ref[...], axis=0)
    sum_all = p[0:2]
    sum_pos = p[2:4]
    pos_counts = p[4:6]
    edge_counts = p[6:8]
    bce_sums = p[8:10]
    valid = (pos_counts > 0) & (edge_counts - pos_counts > 0)
    diff = jnp.where(valid, jnp.log(sum_all) - jnp.log(sum_pos), 0.0)
    n_valid = jnp.maximum(jnp.sum(valid.astype(jnp.float32)), 1.0)
    loss_inf = jnp.sum(diff) / n_valid
    valid_b = edge_counts > 0
    per_graph = jnp.where(valid_b, bce_sums / jnp.maximum(edge_counts, 1.0), 0.0)
    n_valid_b = jnp.maximum(jnp.sum(valid_b.astype(jnp.float32)), 1.0)
    loss_bce = jnp.sum(per_graph) / n_valid_b
    out_ref[0, 0] = W_INF * loss_inf + W_BCE * loss_bce


def kernel(logits, targets, edge_batch, num_graphs):
    del num_graphs
    logits = logits.reshape(-1).astype(jnp.float32)
    targets = targets.reshape(-1).astype(jnp.float32)
    edge_batch = edge_batch.reshape(-1).astype(jnp.int32)
    maxes = _max_pass(logits)
    gmax = jnp.full((16,), jnp.max(maxes) * jnp.float32(INV_TEMP), jnp.float32)
    parts = _sum_pass(logits, targets, edge_batch, gmax)
    parts = parts.reshape(NW, ACC_R, ACC_C)
    out = pl.pallas_call(
        _finish_body,
        out_shape=jax.ShapeDtypeStruct((1, 1), jnp.float32),
        out_specs=pl.BlockSpec(memory_space=pltpu.SMEM),
    )(parts)
    return out[0, 0]


# speedup vs baseline: 436.1943x; 1.0903x over previous
import functools

import jax
import jax.numpy as jnp
from jax import lax
from jax.experimental import pallas as pl
from jax.experimental.pallas import tpu as pltpu
from jax.experimental.pallas import tpu_sc as plsc

N = 3_200_000
G = 4096
TEMP = 0.07
INV_TEMP = float(1.0 / TEMP)
W_INF = 1.0
W_BCE = 0.5

NC = 2
NS = 16
NW = NC * NS
CHUNK = N // NW
SUB = 20000
NSUB = CHUNK // SUB
NV = SUB // 16

ACC_R, ACC_C = 10, 2048

_LOG1P_C = (
    0.9999999705406564, -0.499995015167874, 0.333192713425025,
    -0.24844369892463897, 0.19111431080531266, -0.13674769705378478,
    0.07836166801651114, -0.029588507391321184, 0.005253457796589797,
)

_mesh = plsc.VectorSubcoreMesh(core_axis_name="c", subcore_axis_name="s")


def _worker_id():
    return lax.axis_index("s") * NC + lax.axis_index("c")


def _max_body(x_ref, out_ref):
    out_ref[0, 0] = jnp.max(x_ref[...])


@functools.partial(
    pl.kernel,
    mesh=_mesh,
    out_type=jax.ShapeDtypeStruct((NW, ACC_R * ACC_C), jnp.float32),
    scratch_types=[
        pltpu.VMEM((SUB,), jnp.float32),
        pltpu.VMEM((SUB,), jnp.float32),
        pltpu.VMEM((SUB,), jnp.int32),
        pltpu.VMEM((ACC_R * ACC_C,), jnp.float32),
        pltpu.VMEM((16,), jnp.float32),
    ],
    compiler_params=pltpu.CompilerParams(needs_layout_passes=False),
)
def _sum_pass(logits_hbm, targets_hbm, edges_hbm, gmax_hbm, out_hbm,
              lbuf, tbuf, ebuf, acc, gbuf):
    wid = _worker_id()
    base = wid * CHUNK

    z = jnp.zeros((16,), jnp.float32)

    def zero_body(c, _):
        acc[pl.ds(c * 16, 16)] = z
        return 0

    lax.fori_loop(0, (ACC_R * ACC_C) // 16, zero_body, 0)

    pltpu.sync_copy(gmax_hbm, gbuf)
    gmax = gbuf[...]

    ones = jnp.ones((16,), jnp.float32)
    zf = jnp.zeros((16,), jnp.float32)

    def flush(cur_f, a0, a1, a2, a3, a4):
        cur = cur_f.astype(jnp.int32)
        plsc.addupdate_scatter(acc, [cur], a0)
        plsc.addupdate_scatter(acc, [cur + G], a1)
        plsc.addupdate_scatter(acc, [cur + 2 * G], a2)
        plsc.addupdate_scatter(acc, [cur + 3 * G], a3)
        plsc.addupdate_scatter(acc, [cur + 4 * G], a4)

    c15 = jnp.full((16, 1), 15, jnp.int32)
    _dnums = lax.GatherDimensionNumbers(
        offset_dims=(), collapsed_slice_dims=(0,), start_index_map=(0,))

    def _splat_last(x):
        return lax.gather(x, c15, _dnums, (1,),
                          mode=lax.GatherScatterMode.PROMISE_IN_BOUNDS)

    def outer(j, carry):
        off = base + j * SUB
        pltpu.sync_copy(logits_hbm.at[pl.ds(off, SUB)], lbuf)
        pltpu.sync_copy(targets_hbm.at[pl.ds(off, SUB)], tbuf)
        pltpu.sync_copy(edges_hbm.at[pl.ds(off, SUB)], ebuf)

        zero = jnp.float32(0.0)

        def compute(i):
            sl = pl.ds(i * 16, 16)
            l = lbuf[sl]
            t = tbuf[sl]
            seg = jnp.minimum(ebuf[sl], G - 1)
            seg_f = seg.astype(jnp.float32)
            s = l * jnp.float32(INV_TEMP)
            e_all = jnp.exp(s - gmax)
            pos = t > jnp.float32(0.5)
            e_pos = jnp.where(pos, e_all, zero)
            posf = jnp.where(pos, jnp.float32(1.0), zero)
            u = jnp.exp(-jnp.abs(l))
            p = jnp.float32(_LOG1P_C[-1])
            for c in _LOG1P_C[-2::-1]:
                p = p * u + jnp.float32(c)
            bce = jnp.maximum(l, jnp.float32(0.0)) - l * t + u * p
            return seg, seg_f, (e_all, e_pos, posf, ones, bce)

        UNROLL = 5

        def inner(i, carry):
            cur_f, *accs = carry
            datas = [compute(UNROLL * i + k) for k in range(UNROLL)]
            cur_new = _splat_last(datas[-1][1])
            chg = cur_new != cur_f
            cur_i = cur_f.astype(jnp.int32)
            aq1s = list(accs)
            news = [zf] * 5
            for seg, seg_f, vals in datas:
                m_prev = seg_f == cur_f
                m_new = seg_f == cur_new
                mid = (~m_prev) & (~m_new)
                for q in range(5):
                    aq1s[q] = aq1s[q] + jnp.where(m_prev, vals[q], zero)
                    news[q] = news[q] + jnp.where(m_new, vals[q], zero)
                    plsc.addupdate_scatter(acc, [seg + q * G], vals[q], mask=mid)
            outs = []
            for q in range(5):
                plsc.addupdate_scatter(acc, [cur_i + q * G], aq1s[q], mask=chg)
                outs.append(jnp.where(chg, news[q], aq1s[q]))
            return (cur_new, *outs)

        return lax.fori_loop(0, NV // UNROLL, inner, carry)

    cur0 = jnp.zeros((16,), jnp.float32)
    carry = lax.fori_loop(0, NSUB, outer, (cur0, zf, zf, zf, zf, zf))
    flush(*carry)
    pltpu.sync_copy(acc, out_hbm.at[wid])


def _finish_body(parts_ref, out_ref):
    p = jnp.sum(parts_ref[...], axis=0)
    sum_all = p[0:2]
    sum_pos = p[2:4]
    pos_counts = p[4:6]
    edge_counts = p[6:8]
    bce_sums = p[8:10]
    valid = (pos_counts > 0) & (edge_counts - pos_counts > 0)
    diff = jnp.where(valid, jnp.log(sum_all) - jnp.log(sum_pos), 0.0)
    n_valid = jnp.maximum(jnp.sum(valid.astype(jnp.float32)), 1.0)
    loss_inf = jnp.sum(diff) / n_valid
    valid_b = edge_counts > 0
    per_graph = jnp.where(valid_b, bce_sums / jnp.maximum(edge_counts, 1.0), 0.0)
    n_valid_b = jnp.maximum(jnp.sum(valid_b.astype(jnp.float32)), 1.0)
    loss_bce = jnp.sum(per_graph) / n_valid_b
    out_ref[0, 0] = W_INF * loss_inf + W_BCE * loss_bce


def kernel(logits, targets, edge_batch, num_graphs):
    del num_graphs
    logits = logits.reshape(-1).astype(jnp.float32)
    targets = targets.reshape(-1).astype(jnp.float32)
    edge_batch = edge_batch.reshape(-1).astype(jnp.int32)
    lmax = pl.pallas_call(
        _max_body,
        out_shape=jax.ShapeDtypeStruct((1, 1), jnp.float32),
        out_specs=pl.BlockSpec(memory_space=pltpu.SMEM),
    )(logits.reshape(1000, 3200))
    gmax = jnp.full((16,), lmax[0, 0] * jnp.float32(INV_TEMP), jnp.float32)
    parts = _sum_pass(logits, targets, edge_batch, gmax)
    parts = parts.reshape(NW, ACC_R, ACC_C)
    out = pl.pallas_call(
        _finish_body,
        out_shape=jax.ShapeDtypeStruct((1, 1), jnp.float32),
        out_specs=pl.BlockSpec(memory_space=pltpu.SMEM),
    )(parts)
    return out[0, 0]
